# baseline (speedup 1.0000x reference)
import functools

import jax
import jax.numpy as jnp
from jax import lax
from jax.experimental import pallas as pl
from jax.experimental.pallas import tpu as pltpu
from jax.experimental.pallas import tpu_sc as plsc


_R = 16


def kernel(x, pe_table):
    B, S, D = x.shape
    N = B * S
    xf = x.reshape(N, D)
    info = plsc.get_sparse_core_info()
    nc, ns = info.num_cores, info.num_subcores
    nw = nc * ns
    rows = N // nw
    nchunk = rows // _R
    nvec = D // 16
    mesh = plsc.VectorSubcoreMesh(core_axis_name="c", subcore_axis_name="s")

    @functools.partial(
        pl.kernel,
        mesh=mesh,
        out_type=jax.ShapeDtypeStruct((N, D), jnp.float32),
        scratch_types=[
            pltpu.VMEM((_R, D), jnp.float32),
            pltpu.VMEM((_R, D), jnp.float32),
            pltpu.VMEM((_R, D), jnp.float32),
        ],
    )
    def sc_add(x_hbm, pe_hbm, out_hbm, xv, pev, ov):
        wid = lax.axis_index("s") * nc + lax.axis_index("c")
        base = wid * rows
        pe_base = base % S

        def chunk(i, carry):
            row0 = base + i * _R
            prow0 = pe_base + i * _R
            pltpu.sync_copy(x_hbm.at[pl.ds(row0, _R)], xv)
            pltpu.sync_copy(pe_hbm.at[pl.ds(prow0, _R)], pev)

            def row(r, c):
                def vec(j, cc):
                    o = j * 16
                    ov[r, pl.ds(o, 16)] = xv[r, pl.ds(o, 16)] + pev[r, pl.ds(o, 16)]
                    return cc

                lax.fori_loop(0, nvec, vec, 0)
                return c

            lax.fori_loop(0, _R, row, 0)
            pltpu.sync_copy(ov, out_hbm.at[pl.ds(row0, _R)])
            return carry

        lax.fori_loop(0, nchunk, chunk, 0)

    out = sc_add(xf, pe_table)
    return out.reshape(B, S, D)


# speedup vs baseline: 5.5836x; 5.5836x over previous
import jax
import jax.numpy as jnp
from jax.experimental import pallas as pl
from jax.experimental.pallas import tpu as pltpu


_BS = 2048


def _add_kernel(x_ref, pe_ref, o_ref):
    o_ref[...] = x_ref[...] + pe_ref[...]


_BB = 1


def kernel(x, pe_table):
    B, S, D = x.shape
    grid = (S // _BS, B // _BB)
    return pl.pallas_call(
        _add_kernel,
        grid=grid,
        in_specs=[
            pl.BlockSpec((_BB, _BS, D), lambda s, b: (b, s, 0)),
            pl.BlockSpec((_BS, D), lambda s, b: (s, 0)),
        ],
        out_specs=pl.BlockSpec((_BB, _BS, D), lambda s, b: (b, s, 0)),
        out_shape=jax.ShapeDtypeStruct((B, S, D), x.dtype),
        compiler_params=pltpu.CompilerParams(
            dimension_semantics=("parallel", "parallel"),
            vmem_limit_bytes=63 * 1024 * 1024,
        ),
    )(x, pe_table)
